# baseline (device time: 44997 ns/iter reference)
import jax
import jax.numpy as jnp
from jax import lax
from jax.experimental import pallas as pl
from jax.experimental.pallas import tpu as pltpu

N_DEV = 4
SIGMA_CLIP = 5.5


def kernel(x, w_mat, scale_x, scale_w):
    m_per, k_dim = x.shape
    _, n_dim = w_mat.shape
    n_per = n_dim // N_DEV
    m_tot = m_per * N_DEV

    my = lax.axis_index("i")
    offs = jnp.array([1, 2, 3, 0], dtype=jnp.int32)
    perm = jnp.remainder(my.astype(jnp.int32) + offs, N_DEV)
    sigma = k_dim ** 0.5

    def body(perm_ref, x_hbm, w_ref, sx_ref, sw_ref, out_ref,
             x8_ref, xstage, qsend, qrecv, load_sem, send_sems, recv_sems):
        j = pl.program_id(0)
        my_pos = lax.axis_index("i")
        scale = sx_ref[0] * sw_ref[0]
        qs = scale * (SIGMA_CLIP * sigma / 127.0)
        inv_qs = 1.0 / qs

        @pl.when(j == 0)
        def _():
            dma = pltpu.make_async_copy(x_hbm, xstage, load_sem)
            dma.start()
            dma.wait()
            x8_ref[...] = xstage[...].astype(jnp.float8_e4m3fn)

        blk = lax.dot_general(
            x8_ref[...], w_ref[...].astype(jnp.float8_e4m3fn),
            (((1,), (0,)), ((), ())),
            preferred_element_type=jnp.float32,
        ) * scale

        @pl.when(j < N_DEV - 1)
        def _():
            q = jnp.clip(jnp.round(blk * inv_qs), -127.0, 127.0)
            qsend[j] = q.astype(jnp.int8)
            dest = perm_ref[j]
            rdma = pltpu.make_async_remote_copy(
                src_ref=qsend.at[j],
                dst_ref=qrecv.at[j],
                send_sem=send_sems.at[j],
                recv_sem=recv_sems.at[j],
                device_id=(dest,),
                device_id_type=pl.DeviceIdType.MESH,
            )
            rdma.start()

        @pl.when(j == N_DEV - 1)
        def _():
            out_ref[pl.ds(my_pos * m_per, m_per), :] = blk.astype(jnp.bfloat16)
            for jj in range(N_DEV - 1):
                dest = perm_ref[jj]
                src_dev = lax.rem(my_pos - 1 - jj + N_DEV, N_DEV)
                d = pltpu.make_async_remote_copy(
                    src_ref=qsend.at[jj],
                    dst_ref=qrecv.at[jj],
                    send_sem=send_sems.at[jj],
                    recv_sem=recv_sems.at[jj],
                    device_id=(dest,),
                    device_id_type=pl.DeviceIdType.MESH,
                )
                d.wait_send()
                d.wait_recv()
                deq = qrecv[jj].astype(jnp.float32) * qs
                out_ref[pl.ds(src_dev * m_per, m_per), :] = deq.astype(jnp.bfloat16)

    grid_spec = pltpu.PrefetchScalarGridSpec(
        num_scalar_prefetch=1,
        grid=(N_DEV,),
        in_specs=[
            pl.BlockSpec(memory_space=pltpu.MemorySpace.HBM),
            pl.BlockSpec((k_dim, n_per), lambda j, p: (0, p[j])),
            pl.BlockSpec(memory_space=pltpu.SMEM),
            pl.BlockSpec(memory_space=pltpu.SMEM),
        ],
        out_specs=pl.BlockSpec((m_tot, n_per), lambda j, p: (0, 0)),
        scratch_shapes=[
            pltpu.VMEM((m_per, k_dim), jnp.float8_e4m3fn),
            pltpu.VMEM((m_per, k_dim), jnp.float32),
            pltpu.VMEM((N_DEV - 1, m_per, n_per), jnp.int8),
            pltpu.VMEM((N_DEV - 1, m_per, n_per), jnp.int8),
            pltpu.SemaphoreType.DMA,
            pltpu.SemaphoreType.DMA((N_DEV - 1,)),
            pltpu.SemaphoreType.DMA((N_DEV - 1,)),
        ],
    )

    return pl.pallas_call(
        body,
        grid_spec=grid_spec,
        out_shape=jax.ShapeDtypeStruct((m_tot, n_per), jnp.bfloat16),
        compiler_params=pltpu.CompilerParams(
            vmem_limit_bytes=56 * 1024 * 1024,
        ),
    )(perm, x, w_mat, scale_x, scale_w)


# device time: 38480 ns/iter; 1.1694x vs baseline; 1.1694x over previous
import jax
import jax.numpy as jnp
from jax import lax
from jax.experimental import pallas as pl
from jax.experimental.pallas import tpu as pltpu

N_DEV = 4
SIGMA_CLIP = 5.5


def kernel(x, w_mat, scale_x, scale_w):
    m_per, k_dim = x.shape
    _, n_dim = w_mat.shape
    n_per = n_dim // N_DEV
    m_tot = m_per * N_DEV

    my = lax.axis_index("i")
    offs = jnp.array([1, 2, 3, 0], dtype=jnp.int32)
    perm = jnp.remainder(my.astype(jnp.int32) + offs, N_DEV)
    sigma = k_dim ** 0.5

    def body(perm_ref, x_ref, w_ref, sx_ref, sw_ref, out_ref,
             x8_ref, qsend, qrecv, send_sems, recv_sems):
        j = pl.program_id(0)
        my_pos = lax.axis_index("i")
        scale = sx_ref[0] * sw_ref[0]
        qs = scale * (SIGMA_CLIP * sigma / 127.0)
        inv_qs = 1.0 / qs

        @pl.when(j == 0)
        def _():
            x8_ref[...] = x_ref[...].astype(jnp.float8_e4m3fn)

        blk = lax.dot_general(
            x8_ref[...], w_ref[...].astype(jnp.float8_e4m3fn),
            (((1,), (0,)), ((), ())),
            preferred_element_type=jnp.float32,
        ) * scale

        @pl.when(j < N_DEV - 1)
        def _():
            q = jnp.clip(jnp.round(blk * inv_qs), -127.0, 127.0)
            qsend[j] = q.astype(jnp.int8)
            dest = perm_ref[j]
            rdma = pltpu.make_async_remote_copy(
                src_ref=qsend.at[j],
                dst_ref=qrecv.at[j],
                send_sem=send_sems.at[j],
                recv_sem=recv_sems.at[j],
                device_id=(dest,),
                device_id_type=pl.DeviceIdType.MESH,
            )
            rdma.start()

        @pl.when(j == N_DEV - 1)
        def _():
            out_ref[pl.ds(my_pos * m_per, m_per), :] = blk.astype(jnp.bfloat16)
            for jj in range(N_DEV - 1):
                dest = perm_ref[jj]
                src_dev = lax.rem(my_pos - 1 - jj + N_DEV, N_DEV)
                d = pltpu.make_async_remote_copy(
                    src_ref=qsend.at[jj],
                    dst_ref=qrecv.at[jj],
                    send_sem=send_sems.at[jj],
                    recv_sem=recv_sems.at[jj],
                    device_id=(dest,),
                    device_id_type=pl.DeviceIdType.MESH,
                )
                d.wait_send()
                d.wait_recv()
                deq = qrecv[jj].astype(jnp.float32) * qs
                out_ref[pl.ds(src_dev * m_per, m_per), :] = deq.astype(jnp.bfloat16)

    grid_spec = pltpu.PrefetchScalarGridSpec(
        num_scalar_prefetch=1,
        grid=(N_DEV,),
        in_specs=[
            pl.BlockSpec((m_per, k_dim), lambda j, p: (0, 0)),
            pl.BlockSpec((k_dim, n_per), lambda j, p: (0, p[j])),
            pl.BlockSpec(memory_space=pltpu.SMEM),
            pl.BlockSpec(memory_space=pltpu.SMEM),
        ],
        out_specs=pl.BlockSpec((m_tot, n_per), lambda j, p: (0, 0)),
        scratch_shapes=[
            pltpu.VMEM((m_per, k_dim), jnp.float8_e4m3fn),
            pltpu.VMEM((N_DEV - 1, m_per, n_per), jnp.int8),
            pltpu.VMEM((N_DEV - 1, m_per, n_per), jnp.int8),
            pltpu.SemaphoreType.DMA((N_DEV - 1,)),
            pltpu.SemaphoreType.DMA((N_DEV - 1,)),
        ],
    )

    return pl.pallas_call(
        body,
        grid_spec=grid_spec,
        out_shape=jax.ShapeDtypeStruct((m_tot, n_per), jnp.bfloat16),
    )(perm, x, w_mat, scale_x, scale_w)
